# baseline (speedup 1.0000x reference)
import functools

import jax
import jax.numpy as jnp
from jax import lax
from jax.experimental import pallas as pl
from jax.experimental.pallas import tpu as pltpu
from jax.experimental.pallas import tpu_sc as plsc

N = 10000
NPAD = 10240
E = 320000
H = 128
HH = H // 2
D_EDGE = 16
MP = 5

TRE = 3200
TRN = 2048

NW = 32
PW = E // NW
CH = 40
NCH = PW // CH
PWS = E // 16
CHS = 80
NCHS = PWS // CHS
NB = 5
RPT = NPAD // 16


def _ln(xx, g, beta):
    m = jnp.mean(xx, axis=-1, keepdims=True)
    v = jnp.mean((xx - m) ** 2, axis=-1, keepdims=True)
    return (xx - m) * lax.rsqrt(v + 1e-5) * g + beta


def _dot(a, b):
    return jnp.dot(a, b, preferred_element_type=jnp.float32)


def _bn_stats_body(x_ref, o_ref):
    i = pl.program_id(0)
    xb = x_ref[...]
    part = jnp.concatenate(
        [jnp.sum(xb, axis=0, keepdims=True),
         jnp.sum(xb * xb, axis=0, keepdims=True)], axis=0)

    @pl.when(i == 0)
    def _():
        o_ref[...] = part

    @pl.when(i > 0)
    def _():
        o_ref[...] += part


_bn_stats = pl.pallas_call(
    _bn_stats_body,
    grid=(E // TRE,),
    in_specs=[pl.BlockSpec((TRE, D_EDGE), lambda i: (i, 0))],
    out_specs=pl.BlockSpec((2, D_EDGE), lambda i: (0, 0)),
    out_shape=jax.ShapeDtypeStruct((2, D_EDGE), jnp.float32),
)


def _w_specs(shapes):
    return [pl.BlockSpec(s, lambda i: tuple(0 for _ in s)) for s in shapes]


def _enc_edge_body(x_ref, w1, b1, w2, b2, w3, b3, g, beta, o_ref):
    l1 = jax.nn.relu(_dot(x_ref[...], w1[...]) + b1[...])
    l2 = jax.nn.relu(_dot(l1, w2[...]) + b2[...])
    l3 = _dot(l2, w3[...]) + b3[...]
    o_ref[...] = _ln(l3, g[...], beta[...])


_enc_edge = pl.pallas_call(
    _enc_edge_body,
    grid=(E // TRE,),
    in_specs=[pl.BlockSpec((TRE, D_EDGE), lambda i: (i, 0))]
    + _w_specs([(D_EDGE, H), (1, H), (H, H), (1, H), (H, H), (1, H), (1, H),
                (1, H)]),
    out_specs=pl.BlockSpec((TRE, H), lambda i: (i, 0)),
    out_shape=jax.ShapeDtypeStruct((E, H), jnp.float32),
)


def _enc_node_body(x_ref, w1, b1, w2, b2, w3, b3, g, beta, wa, wb,
                   h_ref, ha_ref, hb_ref):
    l1 = jax.nn.relu(_dot(x_ref[...], w1[...]) + b1[...])
    l2 = jax.nn.relu(_dot(l1, w2[...]) + b2[...])
    l3 = _dot(l2, w3[...]) + b3[...]
    h = _ln(l3, g[...], beta[...])
    h_ref[...] = h
    ha_ref[...] = _dot(h, wa[...])
    hb_ref[...] = _dot(h, wb[...])


_enc_node = pl.pallas_call(
    _enc_node_body,
    grid=(NPAD // TRN,),
    in_specs=[pl.BlockSpec((TRN, H), lambda i: (i, 0))]
    + _w_specs([(H, H), (1, H), (H, H), (1, H), (H, H), (1, H), (1, H), (1, H),
                (H, H), (H, H)]),
    out_specs=[pl.BlockSpec((TRN, H), lambda i: (i, 0))] * 3,
    out_shape=[jax.ShapeDtypeStruct((NPAD, H), jnp.float32)] * 3,
)


def _edge_mlp_body(ga_ref, gb_ref, e_ref, w1c, b1, w2, b2, w3, b3, g, beta, wm,
                   e_out, m_out):
    e_in = e_ref[...]
    l1 = jax.nn.relu(ga_ref[...] + gb_ref[...] + _dot(e_in, w1c[...]) + b1[...])
    l2 = jax.nn.relu(_dot(l1, w2[...]) + b2[...])
    l3 = _dot(l2, w3[...]) + b3[...]
    e_new = _ln(l3, g[...], beta[...]) + e_in
    e_out[...] = e_new
    mm = _dot(e_new, wm[...])
    m_out[0] = mm[:, :HH]
    m_out[1] = mm[:, HH:]


_edge_mlp = pl.pallas_call(
    _edge_mlp_body,
    grid=(E // TRE,),
    in_specs=[pl.BlockSpec((TRE, H), lambda i: (i, 0))] * 3
    + _w_specs([(H, H), (1, H), (H, H), (1, H), (H, H), (1, H), (1, H), (1, H),
                (H, H)]),
    out_specs=[pl.BlockSpec((TRE, H), lambda i: (i, 0)),
               pl.BlockSpec((2, TRE, HH), lambda i: (0, i, 0))],
    out_shape=[jax.ShapeDtypeStruct((E, H), jnp.float32),
               jax.ShapeDtypeStruct((2, E, HH), jnp.float32)],
)


_AGG_SPECS = [pl.BlockSpec((1, TRN, HH), lambda i: (0, i, 0)),
              pl.BlockSpec((1, TRN, HH), lambda i: (1, i, 0))]


def _node_mlp_body(h_ref, a0_ref, a1_ref, w1, b1, w2, b2, w3, b3, g, beta,
                   wa, wb, h_out, ha_out, hb_out):
    h_in = h_ref[...]
    agg = jnp.concatenate([a0_ref[0], a1_ref[0]], axis=-1)
    l1 = jax.nn.relu(_dot(h_in, w1[...]) + agg + b1[...])
    l2 = jax.nn.relu(_dot(l1, w2[...]) + b2[...])
    l3 = _dot(l2, w3[...]) + b3[...]
    h_new = _ln(l3, g[...], beta[...]) + h_in
    h_out[...] = h_new
    ha_out[...] = _dot(h_new, wa[...])
    hb_out[...] = _dot(h_new, wb[...])


_node_mlp = pl.pallas_call(
    _node_mlp_body,
    grid=(NPAD // TRN,),
    in_specs=[pl.BlockSpec((TRN, H), lambda i: (i, 0))] + _AGG_SPECS
    + _w_specs([(H, H), (1, H), (H, H), (1, H), (H, H), (1, H), (1, H), (1, H),
                (H, H), (H, H)]),
    out_specs=[pl.BlockSpec((TRN, H), lambda i: (i, 0))] * 3,
    out_shape=[jax.ShapeDtypeStruct((NPAD, H), jnp.float32)] * 3,
)


def _node_last_body(h_ref, a0_ref, a1_ref, w1, b1, w2, b2, w3, b3, g, beta,
                    h_out):
    h_in = h_ref[...]
    agg = jnp.concatenate([a0_ref[0], a1_ref[0]], axis=-1)
    l1 = jax.nn.relu(_dot(h_in, w1[...]) + agg + b1[...])
    l2 = jax.nn.relu(_dot(l1, w2[...]) + b2[...])
    l3 = _dot(l2, w3[...]) + b3[...]
    h_out[...] = _ln(l3, g[...], beta[...]) + h_in


_node_last = pl.pallas_call(
    _node_last_body,
    grid=(NPAD // TRN,),
    in_specs=[pl.BlockSpec((TRN, H), lambda i: (i, 0))] + _AGG_SPECS
    + _w_specs([(H, H), (1, H), (H, H), (1, H), (H, H), (1, H), (1, H),
                (1, H)]),
    out_specs=pl.BlockSpec((TRN, H), lambda i: (i, 0)),
    out_shape=jax.ShapeDtypeStruct((NPAD, H), jnp.float32),
)


def _dec_body(h_ref, x_ref, w1, b1, w2, b2, w3, b3, o_ref):
    l1 = jax.nn.relu(_dot(h_ref[...], w1[...]) + b1[...])
    l2 = jax.nn.relu(_dot(l1, w2[...]) + b2[...])
    l3 = _dot(l2, w3[...]) + b3[...]
    o_ref[...] = l3 * 0.005 + x_ref[:, :3]


_dec = pl.pallas_call(
    _dec_body,
    grid=(NPAD // TRN,),
    in_specs=[pl.BlockSpec((TRN, H), lambda i: (i, 0)),
              pl.BlockSpec((TRN, H), lambda i: (i, 0))]
    + _w_specs([(H, H), (1, H), (H, H), (1, H), (H, 3), (1, 3)]),
    out_specs=pl.BlockSpec((TRN, 3), lambda i: (i, 0)),
    out_shape=jax.ShapeDtypeStruct((NPAD, 3), jnp.float32),
)


_sc_mesh = plsc.VectorSubcoreMesh(core_axis_name="c", subcore_axis_name="s")


@functools.partial(
    pl.kernel,
    out_type=[jax.ShapeDtypeStruct((E, H), jnp.float32),
              jax.ShapeDtypeStruct((E, H), jnp.float32)],
    mesh=_sc_mesh,
    scratch_types=[pltpu.VMEM((NCH, CH), jnp.int32),
                   pltpu.VMEM((NCH, CH), jnp.int32),
                   pltpu.VMEM((NB, CH, H), jnp.float32),
                   pltpu.VMEM((NB, CH, H), jnp.float32),
                   pltpu.SemaphoreType.DMA((NB,)),
                   pltpu.SemaphoreType.DMA((NB,)),
                   pltpu.SemaphoreType.DMA((NB,)),
                   pltpu.SemaphoreType.DMA((NB,))],
)
def _sc_gather(ha_hbm, hb_hbm, row3_hbm, col3_hbm, ga_hbm, gb_hbm,
               idx_a, idx_b, buf_a, buf_b, gsa, gsb, wsa, wsb):
    cid = lax.axis_index("c")
    sid = lax.axis_index("s")
    wid = sid * 2 + cid
    base = wid * PW
    pltpu.sync_copy(row3_hbm.at[wid], idx_a)
    pltpu.sync_copy(col3_hbm.at[wid], idx_b)
    for b in range(NB):
        pltpu.async_copy(ha_hbm.at[idx_a.at[b]], buf_a.at[b], gsa.at[b])
        pltpu.async_copy(hb_hbm.at[idx_b.at[b]], buf_b.at[b], gsb.at[b])

    @pl.loop(0, NCH, step=NB)
    def _round(step):
        for b in range(NB):
            ci = step + b
            s = base + ci * CH
            pltpu.make_async_copy(ha_hbm.at[idx_a.at[ci]], buf_a.at[b],
                                  gsa.at[b]).wait()
            pltpu.make_async_copy(hb_hbm.at[idx_b.at[ci]], buf_b.at[b],
                                  gsb.at[b]).wait()
            pltpu.async_copy(buf_a.at[b], ga_hbm.at[pl.ds(s, CH)], wsa.at[b])
            pltpu.async_copy(buf_b.at[b], gb_hbm.at[pl.ds(s, CH)], wsb.at[b])
        for b in range(NB):
            cj = step + NB + b
            pltpu.make_async_copy(buf_a.at[b], ga_hbm.at[pl.ds(base, CH)],
                                  wsa.at[b]).wait()
            pltpu.make_async_copy(buf_b.at[b], gb_hbm.at[pl.ds(base, CH)],
                                  wsb.at[b]).wait()

            @pl.when(cj < NCH)
            def _():
                pltpu.async_copy(ha_hbm.at[idx_a.at[cj]], buf_a.at[b],
                                 gsa.at[b])
                pltpu.async_copy(hb_hbm.at[idx_b.at[cj]], buf_b.at[b],
                                 gsb.at[b])


@functools.partial(
    pl.kernel,
    out_type=jax.ShapeDtypeStruct((2, NPAD, HH), jnp.float32),
    mesh=_sc_mesh,
    scratch_types=[pltpu.VMEM_SHARED((NPAD, HH), jnp.float32),
                   pltpu.VMEM((NCHS, CHS), jnp.int32),
                   pltpu.VMEM((NB, CHS, HH), jnp.float32),
                   pltpu.SemaphoreType.DMA((NB,))],
)
def _sc_scatter(ms_hbm, cols_hbm, zeros_hbm, agg_hbm, shared, idx, buf, lsem):
    cid = lax.axis_index("c")
    sid = lax.axis_index("s")
    base = sid * PWS
    pltpu.sync_copy(zeros_hbm.at[pl.ds(sid * RPT, RPT)],
                    shared.at[pl.ds(sid * RPT, RPT)])
    pltpu.sync_copy(cols_hbm.at[sid], idx)
    plsc.subcore_barrier()
    for b in range(NB):
        pltpu.async_copy(ms_hbm.at[cid, pl.ds(base + b * CHS, CHS)],
                         buf.at[b], lsem.at[b])

    @pl.loop(0, NCHS, step=NB)
    def _round(step):
        for b in range(NB):
            ci = step + b
            cj = ci + NB
            pltpu.make_async_copy(ms_hbm.at[cid, pl.ds(base, CHS)], buf.at[b],
                                  lsem.at[b]).wait()
            pltpu.sync_copy(buf.at[b], shared.at[idx.at[ci]], add=True)

            @pl.when(cj < NCHS)
            def _():
                pltpu.async_copy(ms_hbm.at[cid, pl.ds(base + cj * CHS, CHS)],
                                 buf.at[b], lsem.at[b])

    plsc.subcore_barrier()
    pltpu.sync_copy(shared.at[pl.ds(sid * RPT, RPT)],
                    agg_hbm.at[cid, pl.ds(sid * RPT, RPT)])


def _r(v):
    return v.reshape(1, -1)


def kernel(x, edge_index, edge_attr, params):
    row3 = edge_index[0].reshape(NW, NCH, CH)
    col3 = edge_index[1].reshape(NW, NCH, CH)
    cols = edge_index[1].reshape(16, NCHS, CHS)
    xp = jnp.pad(x, ((0, NPAD - N), (0, 0)))
    zeros_n = jnp.zeros((NPAD, HH), jnp.float32)

    sums = _bn_stats(edge_attr)
    mu = sums[0] / E
    var = sums[1] / E - mu * mu
    s = params["bn"]["gamma"] * lax.rsqrt(var + 1e-5)
    ep = params["edge_enc"]
    w1p = ep["W"][0] * s[:, None]
    b1p = ep["b"][0] + (params["bn"]["beta"] - mu * s) @ ep["W"][0]
    e = _enc_edge(edge_attr, w1p, _r(b1p), ep["W"][1], _r(ep["b"][1]),
                  ep["W"][2], _r(ep["b"][2]), _r(ep["g"]), _r(ep["beta"]))

    np_ = params["node_enc"]
    we0 = params["blocks"][0]["edge"]["W"][0]
    h, ha, hb = _enc_node(xp, np_["W"][0], _r(np_["b"][0]),
                          np_["W"][1], _r(np_["b"][1]),
                          np_["W"][2], _r(np_["b"][2]),
                          _r(np_["g"]), _r(np_["beta"]),
                          we0[:H], we0[H:2 * H])

    for i in range(MP):
        blk = params["blocks"][i]
        pe, pn = blk["edge"], blk["node"]
        ga, gb = _sc_gather(ha, hb, row3, col3)
        e, ms = _edge_mlp(ga, gb, e, pe["W"][0][2 * H:], _r(pe["b"][0]),
                          pe["W"][1], _r(pe["b"][1]), pe["W"][2],
                          _r(pe["b"][2]), _r(pe["g"]), _r(pe["beta"]),
                          pn["W"][0][H:])
        agg2 = _sc_scatter(ms, cols, zeros_n)
        nargs = (h, agg2, agg2, pn["W"][0][:H], _r(pn["b"][0]),
                 pn["W"][1], _r(pn["b"][1]), pn["W"][2], _r(pn["b"][2]),
                 _r(pn["g"]), _r(pn["beta"]))
        if i < MP - 1:
            wen = params["blocks"][i + 1]["edge"]["W"][0]
            h, ha, hb = _node_mlp(*nargs, wen[:H], wen[H:2 * H])
        else:
            h = _node_last(*nargs)

    d = params["dec"]
    out = _dec(h, xp, d["W"][0], _r(d["b"][0]), d["W"][1], _r(d["b"][1]),
               d["W"][2], _r(d["b"][2]))
    return out[:N]


# speedup vs baseline: 1.0003x; 1.0003x over previous
import functools

import jax
import jax.numpy as jnp
from jax import lax
from jax.experimental import pallas as pl
from jax.experimental.pallas import tpu as pltpu
from jax.experimental.pallas import tpu_sc as plsc

N = 10000
NPAD = 10240
E = 320000
H = 128
HH = H // 2
D_EDGE = 16
MP = 5

TRE = 1600
TRN = 1024

NW = 32
PW = E // NW
CH = 40
NCH = PW // CH
PWS = E // 16
CHS = 80
NCHS = PWS // CHS
NB = 5
RPT = NPAD // 16


def _ln(xx, g, beta):
    m = jnp.mean(xx, axis=-1, keepdims=True)
    v = jnp.mean((xx - m) ** 2, axis=-1, keepdims=True)
    return (xx - m) * lax.rsqrt(v + 1e-5) * g + beta


def _dot(a, b):
    return jnp.dot(a, b, preferred_element_type=jnp.float32)


def _bn_stats_body(x_ref, o_ref):
    i = pl.program_id(0)
    xb = x_ref[...]
    part = jnp.concatenate(
        [jnp.sum(xb, axis=0, keepdims=True),
         jnp.sum(xb * xb, axis=0, keepdims=True)], axis=0)

    @pl.when(i == 0)
    def _():
        o_ref[...] = part

    @pl.when(i > 0)
    def _():
        o_ref[...] += part


_bn_stats = pl.pallas_call(
    _bn_stats_body,
    grid=(E // TRE,),
    in_specs=[pl.BlockSpec((TRE, D_EDGE), lambda i: (i, 0))],
    out_specs=pl.BlockSpec((2, D_EDGE), lambda i: (0, 0)),
    out_shape=jax.ShapeDtypeStruct((2, D_EDGE), jnp.float32),
)


def _w_specs(shapes):
    return [pl.BlockSpec(s, lambda i: tuple(0 for _ in s)) for s in shapes]


def _enc_edge_body(x_ref, w1, b1, w2, b2, w3, b3, g, beta, o_ref):
    l1 = jax.nn.relu(_dot(x_ref[...], w1[...]) + b1[...])
    l2 = jax.nn.relu(_dot(l1, w2[...]) + b2[...])
    l3 = _dot(l2, w3[...]) + b3[...]
    o_ref[...] = _ln(l3, g[...], beta[...])


_enc_edge = pl.pallas_call(
    _enc_edge_body,
    grid=(E // TRE,),
    in_specs=[pl.BlockSpec((TRE, D_EDGE), lambda i: (i, 0))]
    + _w_specs([(D_EDGE, H), (1, H), (H, H), (1, H), (H, H), (1, H), (1, H),
                (1, H)]),
    out_specs=pl.BlockSpec((TRE, H), lambda i: (i, 0)),
    out_shape=jax.ShapeDtypeStruct((E, H), jnp.float32),
)


def _enc_node_body(x_ref, w1, b1, w2, b2, w3, b3, g, beta, wa, wb,
                   h_ref, ha_ref, hb_ref):
    l1 = jax.nn.relu(_dot(x_ref[...], w1[...]) + b1[...])
    l2 = jax.nn.relu(_dot(l1, w2[...]) + b2[...])
    l3 = _dot(l2, w3[...]) + b3[...]
    h = _ln(l3, g[...], beta[...])
    h_ref[...] = h
    ha_ref[...] = _dot(h, wa[...])
    hb_ref[...] = _dot(h, wb[...])


_enc_node = pl.pallas_call(
    _enc_node_body,
    grid=(NPAD // TRN,),
    in_specs=[pl.BlockSpec((TRN, H), lambda i: (i, 0))]
    + _w_specs([(H, H), (1, H), (H, H), (1, H), (H, H), (1, H), (1, H), (1, H),
                (H, H), (H, H)]),
    out_specs=[pl.BlockSpec((TRN, H), lambda i: (i, 0))] * 3,
    out_shape=[jax.ShapeDtypeStruct((NPAD, H), jnp.float32)] * 3,
)


def _edge_mlp_body(ga_ref, gb_ref, e_ref, w1c, b1, w2, b2, w3, b3, g, beta, wm,
                   e_out, m_out):
    e_in = e_ref[...]
    l1 = jax.nn.relu(ga_ref[...] + gb_ref[...] + _dot(e_in, w1c[...]) + b1[...])
    l2 = jax.nn.relu(_dot(l1, w2[...]) + b2[...])
    l3 = _dot(l2, w3[...]) + b3[...]
    e_new = _ln(l3, g[...], beta[...]) + e_in
    e_out[...] = e_new
    mm = _dot(e_new, wm[...])
    m_out[0] = mm[:, :HH]
    m_out[1] = mm[:, HH:]


_edge_mlp = pl.pallas_call(
    _edge_mlp_body,
    grid=(E // TRE,),
    in_specs=[pl.BlockSpec((TRE, H), lambda i: (i, 0))] * 3
    + _w_specs([(H, H), (1, H), (H, H), (1, H), (H, H), (1, H), (1, H), (1, H),
                (H, H)]),
    out_specs=[pl.BlockSpec((TRE, H), lambda i: (i, 0)),
               pl.BlockSpec((2, TRE, HH), lambda i: (0, i, 0))],
    out_shape=[jax.ShapeDtypeStruct((E, H), jnp.float32),
               jax.ShapeDtypeStruct((2, E, HH), jnp.float32)],
)


_AGG_SPECS = [pl.BlockSpec((1, TRN, HH), lambda i: (0, i, 0)),
              pl.BlockSpec((1, TRN, HH), lambda i: (1, i, 0))]


def _node_mlp_body(h_ref, a0_ref, a1_ref, w1, b1, w2, b2, w3, b3, g, beta,
                   wa, wb, h_out, ha_out, hb_out):
    h_in = h_ref[...]
    agg = jnp.concatenate([a0_ref[0], a1_ref[0]], axis=-1)
    l1 = jax.nn.relu(_dot(h_in, w1[...]) + agg + b1[...])
    l2 = jax.nn.relu(_dot(l1, w2[...]) + b2[...])
    l3 = _dot(l2, w3[...]) + b3[...]
    h_new = _ln(l3, g[...], beta[...]) + h_in
    h_out[...] = h_new
    ha_out[...] = _dot(h_new, wa[...])
    hb_out[...] = _dot(h_new, wb[...])


_node_mlp = pl.pallas_call(
    _node_mlp_body,
    grid=(NPAD // TRN,),
    in_specs=[pl.BlockSpec((TRN, H), lambda i: (i, 0))] + _AGG_SPECS
    + _w_specs([(H, H), (1, H), (H, H), (1, H), (H, H), (1, H), (1, H), (1, H),
                (H, H), (H, H)]),
    out_specs=[pl.BlockSpec((TRN, H), lambda i: (i, 0))] * 3,
    out_shape=[jax.ShapeDtypeStruct((NPAD, H), jnp.float32)] * 3,
)


def _node_last_body(h_ref, a0_ref, a1_ref, w1, b1, w2, b2, w3, b3, g, beta,
                    h_out):
    h_in = h_ref[...]
    agg = jnp.concatenate([a0_ref[0], a1_ref[0]], axis=-1)
    l1 = jax.nn.relu(_dot(h_in, w1[...]) + agg + b1[...])
    l2 = jax.nn.relu(_dot(l1, w2[...]) + b2[...])
    l3 = _dot(l2, w3[...]) + b3[...]
    h_out[...] = _ln(l3, g[...], beta[...]) + h_in


_node_last = pl.pallas_call(
    _node_last_body,
    grid=(NPAD // TRN,),
    in_specs=[pl.BlockSpec((TRN, H), lambda i: (i, 0))] + _AGG_SPECS
    + _w_specs([(H, H), (1, H), (H, H), (1, H), (H, H), (1, H), (1, H),
                (1, H)]),
    out_specs=pl.BlockSpec((TRN, H), lambda i: (i, 0)),
    out_shape=jax.ShapeDtypeStruct((NPAD, H), jnp.float32),
)


def _dec_body(h_ref, x_ref, w1, b1, w2, b2, w3, b3, o_ref):
    l1 = jax.nn.relu(_dot(h_ref[...], w1[...]) + b1[...])
    l2 = jax.nn.relu(_dot(l1, w2[...]) + b2[...])
    l3 = _dot(l2, w3[...]) + b3[...]
    o_ref[...] = l3 * 0.005 + x_ref[:, :3]


_dec = pl.pallas_call(
    _dec_body,
    grid=(NPAD // TRN,),
    in_specs=[pl.BlockSpec((TRN, H), lambda i: (i, 0)),
              pl.BlockSpec((TRN, H), lambda i: (i, 0))]
    + _w_specs([(H, H), (1, H), (H, H), (1, H), (H, 3), (1, 3)]),
    out_specs=pl.BlockSpec((TRN, 3), lambda i: (i, 0)),
    out_shape=jax.ShapeDtypeStruct((NPAD, 3), jnp.float32),
)


_sc_mesh = plsc.VectorSubcoreMesh(core_axis_name="c", subcore_axis_name="s")


@functools.partial(
    pl.kernel,
    out_type=[jax.ShapeDtypeStruct((E, H), jnp.float32),
              jax.ShapeDtypeStruct((E, H), jnp.float32)],
    mesh=_sc_mesh,
    scratch_types=[pltpu.VMEM((NCH, CH), jnp.int32),
                   pltpu.VMEM((NCH, CH), jnp.int32),
                   pltpu.VMEM((NB, CH, H), jnp.float32),
                   pltpu.VMEM((NB, CH, H), jnp.float32),
                   pltpu.SemaphoreType.DMA((NB,)),
                   pltpu.SemaphoreType.DMA((NB,)),
                   pltpu.SemaphoreType.DMA((NB,)),
                   pltpu.SemaphoreType.DMA((NB,))],
)
def _sc_gather(ha_hbm, hb_hbm, row3_hbm, col3_hbm, ga_hbm, gb_hbm,
               idx_a, idx_b, buf_a, buf_b, gsa, gsb, wsa, wsb):
    cid = lax.axis_index("c")
    sid = lax.axis_index("s")
    wid = sid * 2 + cid
    base = wid * PW
    pltpu.sync_copy(row3_hbm.at[wid], idx_a)
    pltpu.sync_copy(col3_hbm.at[wid], idx_b)
    for b in range(NB):
        pltpu.async_copy(ha_hbm.at[idx_a.at[b]], buf_a.at[b], gsa.at[b])
        pltpu.async_copy(hb_hbm.at[idx_b.at[b]], buf_b.at[b], gsb.at[b])

    @pl.loop(0, NCH, step=NB)
    def _round(step):
        for b in range(NB):
            ci = step + b
            s = base + ci * CH
            pltpu.make_async_copy(ha_hbm.at[idx_a.at[ci]], buf_a.at[b],
                                  gsa.at[b]).wait()
            pltpu.make_async_copy(hb_hbm.at[idx_b.at[ci]], buf_b.at[b],
                                  gsb.at[b]).wait()
            pltpu.async_copy(buf_a.at[b], ga_hbm.at[pl.ds(s, CH)], wsa.at[b])
            pltpu.async_copy(buf_b.at[b], gb_hbm.at[pl.ds(s, CH)], wsb.at[b])
        for b in range(NB):
            cj = step + NB + b
            pltpu.make_async_copy(buf_a.at[b], ga_hbm.at[pl.ds(base, CH)],
                                  wsa.at[b]).wait()
            pltpu.make_async_copy(buf_b.at[b], gb_hbm.at[pl.ds(base, CH)],
                                  wsb.at[b]).wait()

            @pl.when(cj < NCH)
            def _():
                pltpu.async_copy(ha_hbm.at[idx_a.at[cj]], buf_a.at[b],
                                 gsa.at[b])
                pltpu.async_copy(hb_hbm.at[idx_b.at[cj]], buf_b.at[b],
                                 gsb.at[b])


@functools.partial(
    pl.kernel,
    out_type=jax.ShapeDtypeStruct((2, NPAD, HH), jnp.float32),
    mesh=_sc_mesh,
    scratch_types=[pltpu.VMEM_SHARED((NPAD, HH), jnp.float32),
                   pltpu.VMEM((NCHS, CHS), jnp.int32),
                   pltpu.VMEM((NB, CHS, HH), jnp.float32),
                   pltpu.SemaphoreType.DMA((NB,))],
)
def _sc_scatter(ms_hbm, cols_hbm, zeros_hbm, agg_hbm, shared, idx, buf, lsem):
    cid = lax.axis_index("c")
    sid = lax.axis_index("s")
    base = sid * PWS
    pltpu.sync_copy(zeros_hbm.at[pl.ds(sid * RPT, RPT)],
                    shared.at[pl.ds(sid * RPT, RPT)])
    pltpu.sync_copy(cols_hbm.at[sid], idx)
    plsc.subcore_barrier()
    for b in range(NB):
        pltpu.async_copy(ms_hbm.at[cid, pl.ds(base + b * CHS, CHS)],
                         buf.at[b], lsem.at[b])

    @pl.loop(0, NCHS, step=NB)
    def _round(step):
        for b in range(NB):
            ci = step + b
            cj = ci + NB
            pltpu.make_async_copy(ms_hbm.at[cid, pl.ds(base, CHS)], buf.at[b],
                                  lsem.at[b]).wait()
            pltpu.sync_copy(buf.at[b], shared.at[idx.at[ci]], add=True)

            @pl.when(cj < NCHS)
            def _():
                pltpu.async_copy(ms_hbm.at[cid, pl.ds(base + cj * CHS, CHS)],
                                 buf.at[b], lsem.at[b])

    plsc.subcore_barrier()
    pltpu.sync_copy(shared.at[pl.ds(sid * RPT, RPT)],
                    agg_hbm.at[cid, pl.ds(sid * RPT, RPT)])


def _r(v):
    return v.reshape(1, -1)


def kernel(x, edge_index, edge_attr, params):
    row3 = edge_index[0].reshape(NW, NCH, CH)
    col3 = edge_index[1].reshape(NW, NCH, CH)
    cols = edge_index[1].reshape(16, NCHS, CHS)
    xp = jnp.pad(x, ((0, NPAD - N), (0, 0)))
    zeros_n = jnp.zeros((NPAD, HH), jnp.float32)

    sums = _bn_stats(edge_attr)
    mu = sums[0] / E
    var = sums[1] / E - mu * mu
    s = params["bn"]["gamma"] * lax.rsqrt(var + 1e-5)
    ep = params["edge_enc"]
    w1p = ep["W"][0] * s[:, None]
    b1p = ep["b"][0] + (params["bn"]["beta"] - mu * s) @ ep["W"][0]
    e = _enc_edge(edge_attr, w1p, _r(b1p), ep["W"][1], _r(ep["b"][1]),
                  ep["W"][2], _r(ep["b"][2]), _r(ep["g"]), _r(ep["beta"]))

    np_ = params["node_enc"]
    we0 = params["blocks"][0]["edge"]["W"][0]
    h, ha, hb = _enc_node(xp, np_["W"][0], _r(np_["b"][0]),
                          np_["W"][1], _r(np_["b"][1]),
                          np_["W"][2], _r(np_["b"][2]),
                          _r(np_["g"]), _r(np_["beta"]),
                          we0[:H], we0[H:2 * H])

    for i in range(MP):
        blk = params["blocks"][i]
        pe, pn = blk["edge"], blk["node"]
        ga, gb = _sc_gather(ha, hb, row3, col3)
        e, ms = _edge_mlp(ga, gb, e, pe["W"][0][2 * H:], _r(pe["b"][0]),
                          pe["W"][1], _r(pe["b"][1]), pe["W"][2],
                          _r(pe["b"][2]), _r(pe["g"]), _r(pe["beta"]),
                          pn["W"][0][H:])
        agg2 = _sc_scatter(ms, cols, zeros_n)
        nargs = (h, agg2, agg2, pn["W"][0][:H], _r(pn["b"][0]),
                 pn["W"][1], _r(pn["b"][1]), pn["W"][2], _r(pn["b"][2]),
                 _r(pn["g"]), _r(pn["beta"]))
        if i < MP - 1:
            wen = params["blocks"][i + 1]["edge"]["W"][0]
            h, ha, hb = _node_mlp(*nargs, wen[:H], wen[H:2 * H])
        else:
            h = _node_last(*nargs)

    d = params["dec"]
    out = _dec(h, xp, d["W"][0], _r(d["b"][0]), d["W"][1], _r(d["b"][1]),
               d["W"][2], _r(d["b"][2]))
    return out[:N]
